# baseline (speedup 1.0000x reference)
import functools

import jax
import jax.numpy as jnp
from jax import lax
from jax.experimental import pallas as pl
from jax.experimental.pallas import tpu as pltpu
from jax.experimental.pallas import tpu_sc as plsc

_INV_SQRT2 = 0.7071067811865476

_NT = 16384
_H = 4096
_LANES = 16
_NC = 2
_NS = 16
_NW = _NC * _NS
_TOK_PER_W = _NT // _NW
_T = 8


def _sc_body(moe_hbm, mlp_hbm, out_hbm, moe_v, mlp_v, sem_in):
    wid = lax.axis_index("s") * _NC + lax.axis_index("c")
    base = wid * _TOK_PER_W

    def chunk_body(ci, carry):
        tok = base + ci * _T
        cp1 = pltpu.async_copy(moe_hbm.at[pl.ds(tok, _T)], moe_v, sem_in)
        cp2 = pltpu.async_copy(mlp_hbm.at[pl.ds(tok, _T)], mlp_v, sem_in)
        cp1.wait()
        cp2.wait()

        for t in range(_T):
            def vec_body(j, c, t=t):
                sl = pl.ds(j * _LANES, _LANES)
                a = moe_v[t, 0, sl]
                b = moe_v[t, 1, sl]
                m = mlp_v[t, sl]
                mlp_v[t, sl] = (a + b + m) * _INV_SQRT2
                return c
            lax.fori_loop(0, _H // _LANES, vec_body, 0)

        pltpu.sync_copy(mlp_v, out_hbm.at[pl.ds(tok, _T)])
        return carry

    lax.fori_loop(0, _TOK_PER_W // _T, chunk_body, 0)


_sc_combine = functools.partial(
    pl.kernel,
    out_type=jax.ShapeDtypeStruct((_NT, _H), jnp.float32),
    mesh=plsc.VectorSubcoreMesh(
        core_axis_name="c", subcore_axis_name="s",
        num_cores=_NC, num_subcores=_NS),
    scratch_types=[
        pltpu.VMEM((_T, 2, _H), jnp.float32),
        pltpu.VMEM((_T, _H), jnp.float32),
        pltpu.SemaphoreType.DMA,
    ],
)(_sc_body)


def kernel(moe_hidden_states, mlp_hidden_states):
    return _sc_combine(moe_hidden_states, mlp_hidden_states)


# speedup vs baseline: 3.4767x; 3.4767x over previous
import functools

import jax
import jax.numpy as jnp
from jax import lax
from jax.experimental import pallas as pl
from jax.experimental.pallas import tpu as pltpu
from jax.experimental.pallas import tpu_sc as plsc

_INV_SQRT2 = 0.7071067811865476

_NT = 16384
_H = 4096
_LANES = 16
_NC = 2
_NS = 16
_NW = _NC * _NS
_TOK_PER_W = _NT // _NW
_T = 2
_NCHUNK = _TOK_PER_W // _T


def _sc_body(moe_hbm, mlp_hbm, out_hbm, moe_v, mlp_v, out_v,
             sem_moe, sem_mlp, sem_out):
    wid = lax.axis_index("s") * _NC + lax.axis_index("c")
    base = wid * _TOK_PER_W

    def in_moe(ci, b):
        tok = base + ci * _T
        return pltpu.make_async_copy(
            moe_hbm.at[pl.ds(tok, _T)], moe_v.at[b], sem_moe.at[b])

    def in_mlp(ci, b):
        tok = base + ci * _T
        return pltpu.make_async_copy(
            mlp_hbm.at[pl.ds(tok, _T)], mlp_v.at[b], sem_mlp.at[b])

    def out_cp(ci, b):
        tok = base + ci * _T
        return pltpu.make_async_copy(
            out_v.at[b], out_hbm.at[pl.ds(tok, _T)], sem_out.at[b])

    for b in range(2):
        in_moe(b, b).start()
        in_mlp(b, b).start()

    @pl.loop(0, _NCHUNK, step=2)
    def _(ci):
        for b in range(2):
            cj = ci + b
            in_moe(cj, b).wait()
            in_mlp(cj, b).wait()

            for t in range(_T):
                @plsc.parallel_loop(0, _H // _LANES, unroll=8)
                def _(j, t=t, b=b):
                    sl = pl.ds(j * _LANES, _LANES)
                    out_v[b, t, sl] = (
                        moe_v[b, t, 0, sl] + moe_v[b, t, 1, sl]
                        + mlp_v[b, t, sl]) * _INV_SQRT2

            @pl.when(cj >= 2)
            def _(cj=cj, b=b):
                out_cp(cj - 2, b).wait()

            out_cp(cj, b).start()

            @pl.when(cj + 2 < _NCHUNK)
            def _(cj=cj, b=b):
                in_moe(cj + 2, b).start()
                in_mlp(cj + 2, b).start()

    for b in range(2):
        out_cp(_NCHUNK - 2 + b, b).wait()


_sc_combine = functools.partial(
    pl.kernel,
    out_type=jax.ShapeDtypeStruct((_NT, _H), jnp.float32),
    mesh=plsc.VectorSubcoreMesh(
        core_axis_name="c", subcore_axis_name="s",
        num_cores=_NC, num_subcores=_NS),
    scratch_types=[
        pltpu.VMEM((2, _T, 2, _H), jnp.float32),
        pltpu.VMEM((2, _T, _H), jnp.float32),
        pltpu.VMEM((2, _T, _H), jnp.float32),
        pltpu.SemaphoreType.DMA((2,)),
        pltpu.SemaphoreType.DMA((2,)),
        pltpu.SemaphoreType.DMA((2,)),
    ],
)(_sc_body)


def kernel(moe_hidden_states, mlp_hidden_states):
    return _sc_combine(moe_hidden_states, mlp_hidden_states)


# speedup vs baseline: 3.5888x; 1.0322x over previous
import functools

import jax
import jax.numpy as jnp
from jax import lax
from jax.experimental import pallas as pl
from jax.experimental.pallas import tpu as pltpu
from jax.experimental.pallas import tpu_sc as plsc

_INV_SQRT2 = 0.7071067811865476

_NT = 16384
_H = 4096
_LANES = 16
_NC = 2
_NS = 16
_NW = _NC * _NS
_TOK_PER_W = _NT // _NW
_T = 2
_NCHUNK = _TOK_PER_W // _T
_NBI = 4
_NBO = 2


def _sc_body(moe_hbm, mlp_hbm, out_hbm, moe_v, mlp_v, out_v,
             sem_moe, sem_mlp, sem_out):
    wid = lax.axis_index("s") * _NC + lax.axis_index("c")
    base = wid * _TOK_PER_W

    def in_moe(ci, b):
        tok = base + ci * _T
        return pltpu.make_async_copy(
            moe_hbm.at[pl.ds(tok, _T)], moe_v.at[b], sem_moe.at[b])

    def in_mlp(ci, b):
        tok = base + ci * _T
        return pltpu.make_async_copy(
            mlp_hbm.at[pl.ds(tok, _T)], mlp_v.at[b], sem_mlp.at[b])

    def out_cp(ci, b):
        tok = base + ci * _T
        return pltpu.make_async_copy(
            out_v.at[b], out_hbm.at[pl.ds(tok, _T)], sem_out.at[b])

    for b in range(_NBI):
        in_moe(b, b).start()
        in_mlp(b, b).start()

    @pl.loop(0, _NCHUNK, step=_NBI)
    def _(ci):
        for b in range(_NBI):
            cj = ci + b
            ob = b % _NBO
            in_moe(cj, b).wait()
            in_mlp(cj, b).wait()

            @pl.when(cj >= _NBO)
            def _(cj=cj, ob=ob):
                out_cp(cj - _NBO, ob).wait()

            for t in range(_T):
                @plsc.parallel_loop(0, _H // _LANES, unroll=8)
                def _(j, t=t, b=b, ob=ob):
                    sl = pl.ds(j * _LANES, _LANES)
                    out_v[ob, t, sl] = (
                        moe_v[b, t, 0, sl] + moe_v[b, t, 1, sl]
                        + mlp_v[b, t, sl]) * _INV_SQRT2

            out_cp(cj, ob).start()

            @pl.when(cj + _NBI < _NCHUNK)
            def _(cj=cj, b=b):
                in_moe(cj + _NBI, b).start()
                in_mlp(cj + _NBI, b).start()

    for b in range(_NBO):
        out_cp(_NCHUNK - _NBO + b, b).wait()


_sc_combine = functools.partial(
    pl.kernel,
    out_type=jax.ShapeDtypeStruct((_NT, _H), jnp.float32),
    mesh=plsc.VectorSubcoreMesh(
        core_axis_name="c", subcore_axis_name="s",
        num_cores=_NC, num_subcores=_NS),
    scratch_types=[
        pltpu.VMEM((_NBI, _T, 2, _H), jnp.float32),
        pltpu.VMEM((_NBI, _T, _H), jnp.float32),
        pltpu.VMEM((_NBO, _T, _H), jnp.float32),
        pltpu.SemaphoreType.DMA((_NBI,)),
        pltpu.SemaphoreType.DMA((_NBI,)),
        pltpu.SemaphoreType.DMA((_NBO,)),
    ],
)(_sc_body)


def kernel(moe_hidden_states, mlp_hidden_states):
    return _sc_combine(moe_hidden_states, mlp_hidden_states)
